# baseline (speedup 1.0000x reference)
import jax
import jax.numpy as jnp
from jax import lax
from jax.experimental import pallas as pl
from jax.experimental.pallas import tpu as pltpu
from jax.experimental.pallas import tpu_sc as plsc

N = 320000
C = 128
M = 50000
NC = 2
NS = 16
NW = NC * NS
R = 8
K = NW * R
VR = 200
KA = M // VR
P = 304
CG = C // 16
BPAD = 280


def _issue(c, a0, feats_hbm, ids_hbm, fbuf, ibuf, semf, semi):
    nominal = a0 + c * P
    base = jnp.minimum(nominal, N - P)
    pltpu.make_async_copy(ids_hbm.at[pl.ds(base, P)],
                          ibuf.at[pl.ds(0, P)], semi).start()
    pltpu.make_async_copy(feats_hbm.at[pl.ds(base, P)], fbuf, semf).start()


def _wait_chunk(feats_hbm, ids_hbm, fbuf, ibuf, semf, semi):
    pltpu.make_async_copy(ids_hbm.at[pl.ds(0, P)],
                          ibuf.at[pl.ds(0, P)], semi).wait()
    pltpu.make_async_copy(feats_hbm.at[pl.ds(0, P)], fbuf, semf).wait()


def _body(feats_hbm, ids_hbm, bounds_hbm, out_hbm,
          acc0, cntv, fbuf0, fbuf1, ibuf0, ibuf1, bnd,
          semf0, semf1, semi0, semi1, semo):
    wid = lax.axis_index("s") * NC + lax.axis_index("c")
    pltpu.sync_copy(bounds_hbm, bnd)
    zf = jnp.zeros((16,), jnp.float32)
    ones = jnp.ones((16,), jnp.float32)
    slots = ((fbuf0, ibuf0, semf0, semi0), (fbuf1, ibuf1, semf1, semi1))
    accs = (acc0,)

    def rp_body(rp, carry0):
      for half in range(2):
        acc = accs[0]
        r = rp * 2 + half
        rid = wid * R + r
        v_base = rid * VR

        @pl.when(rid < KA)
        def _range():
            p0 = bnd[pl.ds(rid, 16)][0]
            p1 = bnd[pl.ds(rid + 1, 16)][0]
            a0 = (p0 // 8) * 8
            nch = jnp.maximum((p1 - a0 + P - 1) // P, 0)

            for b in range(2):
                @pl.when(nch > b)
                def _prime():
                    fb, ib, sf, si = slots[b]
                    _issue(b, a0, feats_hbm, ids_hbm, fb, ib, sf, si)

            @pl.when(r >= 1)
            def _wait_prev_out():
                pltpu.make_async_copy(
                    acc, out_hbm.at[pl.ds(0, VR)], semo).wait()

            def zero_body(v, carry):
                for g in range(CG):
                    acc[v, pl.ds(g * 16, 16)] = zf
                cntv[v] = zf
                return carry

            lax.fori_loop(0, VR, zero_body, 0)

            def pair_body(pp, carry):
                for b in range(2):
                    c = pp * 2 + b
                    fb, ib, sf, si = slots[b]

                    @pl.when(c < nch)
                    def _chunk():
                        _wait_chunk(feats_hbm, ids_hbm, fb, ib, sf, si)
                        nominal = a0 + c * P
                        base = jnp.minimum(nominal, N - P)
                        lo = jnp.maximum(p0, nominal) - base
                        hi = jnp.minimum(p1, nominal + P) - base

                        def pt_body(i, c2):
                            v = ib[pl.ds(i, 16)][0] - v_base
                            for g in range(CG):
                                plsc.addupdate(acc.at[v, pl.ds(g * 16, 16)],
                                               fb[i, pl.ds(g * 16, 16)])
                            plsc.addupdate(cntv.at[v], ones)
                            return c2

                        lo16 = ((lo + 15) // 16) * 16
                        ng = jnp.maximum((hi - lo16) // 16, 0)
                        hi16 = lo16 + ng * 16

                        def grp_body(gidx, c2):
                            j = lo16 + gidx * 16
                            idv = ib[pl.ds(j, 16)] - v_base
                            for ln in range(16):
                                v = idv[ln]
                                for g in range(CG):
                                    plsc.addupdate(
                                        acc.at[v, pl.ds(g * 16, 16)],
                                        fb[j + ln, pl.ds(g * 16, 16)])
                                plsc.addupdate(cntv.at[v], ones)
                            return c2

                        lax.fori_loop(lo, jnp.minimum(lo16, hi), pt_body, 0)
                        lax.fori_loop(0, ng, grp_body, 0)
                        lax.fori_loop(jnp.maximum(hi16, lo), hi, pt_body, 0)

                        @pl.when(c + 2 < nch)
                        def _next():
                            _issue(c + 2, a0, feats_hbm, ids_hbm,
                                   fb, ib, sf, si)
                return carry

            lax.fori_loop(0, (nch + 1) // 2, pair_body, 0)

            def div_body(v, carry):
                inv = 1.0 / jnp.maximum(cntv[v], 1.0)
                for g in range(CG):
                    acc[v, pl.ds(g * 16, 16)] = acc[v, pl.ds(g * 16, 16)] * inv
                return carry

            lax.fori_loop(0, VR, div_body, 0)

            pltpu.make_async_copy(
                acc, out_hbm.at[pl.ds(v_base, VR)], semo).start()
      return carry0

    lax.fori_loop(0, R // 2, rp_body, 0)

    act = jnp.clip(KA - wid * R, 0, R)

    @pl.when(act > 0)
    def _drain():
        pltpu.make_async_copy(
            accs[0], out_hbm.at[pl.ds(0, VR)], semo).wait()


def kernel(feats, segment_ids, num_voxels):
    del num_voxels
    qs = jnp.arange(K + 1, dtype=jnp.int32) * VR
    bounds = jnp.searchsorted(segment_ids, qs).astype(jnp.int32)
    bounds = jnp.concatenate(
        [bounds, jnp.full((BPAD - K - 1,), N, dtype=jnp.int32)])
    mesh = plsc.VectorSubcoreMesh(core_axis_name="c", subcore_axis_name="s")
    run = pl.kernel(
        _body,
        out_type=jax.ShapeDtypeStruct((M, C), jnp.float32),
        mesh=mesh,
        scratch_types=[
            pltpu.VMEM((VR, C), jnp.float32),
            pltpu.VMEM((VR, 16), jnp.float32),
            pltpu.VMEM((P, C), jnp.float32),
            pltpu.VMEM((P, C), jnp.float32),
            pltpu.VMEM((P + 16,), jnp.int32),
            pltpu.VMEM((P + 16,), jnp.int32),
            pltpu.VMEM((BPAD,), jnp.int32),
            pltpu.SemaphoreType.DMA,
            pltpu.SemaphoreType.DMA,
            pltpu.SemaphoreType.DMA,
            pltpu.SemaphoreType.DMA,
            pltpu.SemaphoreType.DMA,
        ],
    )
    return run(feats, segment_ids, bounds)


# speedup vs baseline: 1.2984x; 1.2984x over previous
import jax
import jax.numpy as jnp
from jax import lax
from jax.experimental import pallas as pl
from jax.experimental.pallas import tpu as pltpu
from jax.experimental.pallas import tpu_sc as plsc

N = 320000
C = 128
M = 50000
NC = 2
NS = 16
NW = NC * NS
R = 8
K = NW * R
VR = 200
KA = M // VR
P = 192
CG = C // 16
BPAD = 280


def _issue(c, a0, feats_hbm, ids_hbm, fbuf, ibuf, semf, semi):
    nominal = a0 + c * P
    base = jnp.minimum(nominal, N - P)
    pltpu.make_async_copy(ids_hbm.at[pl.ds(base, P)],
                          ibuf.at[pl.ds(0, P)], semi).start()
    pltpu.make_async_copy(feats_hbm.at[pl.ds(base, P)], fbuf, semf).start()


def _wait_chunk(feats_hbm, ids_hbm, fbuf, ibuf, semf, semi):
    pltpu.make_async_copy(ids_hbm.at[pl.ds(0, P)],
                          ibuf.at[pl.ds(0, P)], semi).wait()
    pltpu.make_async_copy(feats_hbm.at[pl.ds(0, P)], fbuf, semf).wait()


def _body(feats_hbm, ids_hbm, bounds_hbm, out_hbm,
          acc0, acc1, cntv, fbuf0, fbuf1, ibuf0, ibuf1, bnd,
          semf0, semf1, semi0, semi1, semo):
    wid = lax.axis_index("s") * NC + lax.axis_index("c")
    pltpu.sync_copy(bounds_hbm, bnd)
    zf = jnp.zeros((16,), jnp.float32)
    ones = jnp.ones((16,), jnp.float32)
    slots = ((fbuf0, ibuf0, semf0, semi0), (fbuf1, ibuf1, semf1, semi1))
    accs = (acc0, acc1)

    def rp_body(rp, carry0):
      for half in range(2):
        acc = accs[half]
        r = rp * 2 + half
        rid = wid * R + r
        v_base = rid * VR

        @pl.when(rid < KA)
        def _range():
            p0 = bnd[pl.ds(rid, 16)][0]
            p1 = bnd[pl.ds(rid + 1, 16)][0]
            a0 = (p0 // 8) * 8
            nch = jnp.maximum((p1 - a0 + P - 1) // P, 0)

            for b in range(2):
                @pl.when(nch > b)
                def _prime():
                    fb, ib, sf, si = slots[b]
                    _issue(b, a0, feats_hbm, ids_hbm, fb, ib, sf, si)

            @pl.when(r >= 2)
            def _wait_prev_out():
                pltpu.make_async_copy(
                    acc, out_hbm.at[pl.ds(0, VR)], semo).wait()

            def zero_body(v, carry):
                for g in range(CG):
                    acc[v, pl.ds(g * 16, 16)] = zf
                cntv[v] = zf
                return carry

            lax.fori_loop(0, VR, zero_body, 0)

            def pair_body(pp, carry):
                for b in range(2):
                    c = pp * 2 + b
                    fb, ib, sf, si = slots[b]

                    @pl.when(c < nch)
                    def _chunk():
                        _wait_chunk(feats_hbm, ids_hbm, fb, ib, sf, si)
                        nominal = a0 + c * P
                        base = jnp.minimum(nominal, N - P)
                        lo = jnp.maximum(p0, nominal) - base
                        hi = jnp.minimum(p1, nominal + P) - base

                        def pt_body(i, c2):
                            v = ib[pl.ds(i, 16)][0] - v_base
                            for g in range(CG):
                                plsc.addupdate(acc.at[v, pl.ds(g * 16, 16)],
                                               fb[i, pl.ds(g * 16, 16)])
                            plsc.addupdate(cntv.at[v], ones)
                            return c2

                        lo16 = ((lo + 15) // 16) * 16
                        ng = jnp.maximum((hi - lo16) // 16, 0)
                        hi16 = lo16 + ng * 16

                        def grp_body(gidx, c2):
                            j = lo16 + gidx * 16
                            idv = ib[pl.ds(j, 16)] - v_base
                            for ln in range(16):
                                v = idv[ln]
                                for g in range(CG):
                                    plsc.addupdate(
                                        acc.at[v, pl.ds(g * 16, 16)],
                                        fb[j + ln, pl.ds(g * 16, 16)])
                                plsc.addupdate(cntv.at[v], ones)
                            return c2

                        lax.fori_loop(lo, jnp.minimum(lo16, hi), pt_body, 0)
                        lax.fori_loop(0, ng, grp_body, 0)
                        lax.fori_loop(jnp.maximum(hi16, lo), hi, pt_body, 0)

                        @pl.when(c + 2 < nch)
                        def _next():
                            _issue(c + 2, a0, feats_hbm, ids_hbm,
                                   fb, ib, sf, si)
                return carry

            lax.fori_loop(0, (nch + 1) // 2, pair_body, 0)

            def div_body(v, carry):
                inv = 1.0 / jnp.maximum(cntv[v], 1.0)
                for g in range(CG):
                    acc[v, pl.ds(g * 16, 16)] = acc[v, pl.ds(g * 16, 16)] * inv
                return carry

            lax.fori_loop(0, VR, div_body, 0)

            pltpu.make_async_copy(
                acc, out_hbm.at[pl.ds(v_base, VR)], semo).start()
      return carry0

    lax.fori_loop(0, R // 2, rp_body, 0)

    act = jnp.clip(KA - wid * R, 0, R)
    for d in range(2):
        @pl.when(act > d)
        def _drain():
            pltpu.make_async_copy(
                accs[d], out_hbm.at[pl.ds(0, VR)], semo).wait()


def kernel(feats, segment_ids, num_voxels):
    del num_voxels
    qs = jnp.arange(K + 1, dtype=jnp.int32) * VR
    bounds = jnp.searchsorted(segment_ids, qs,
                              method="compare_all").astype(jnp.int32)
    bounds = jnp.concatenate(
        [bounds, jnp.full((BPAD - K - 1,), N, dtype=jnp.int32)])
    mesh = plsc.VectorSubcoreMesh(core_axis_name="c", subcore_axis_name="s")
    run = pl.kernel(
        _body,
        out_type=jax.ShapeDtypeStruct((M, C), jnp.float32),
        mesh=mesh,
        scratch_types=[
            pltpu.VMEM((VR, C), jnp.float32),
            pltpu.VMEM((VR, C), jnp.float32),
            pltpu.VMEM((VR, 16), jnp.float32),
            pltpu.VMEM((P, C), jnp.float32),
            pltpu.VMEM((P, C), jnp.float32),
            pltpu.VMEM((P + 16,), jnp.int32),
            pltpu.VMEM((P + 16,), jnp.int32),
            pltpu.VMEM((BPAD,), jnp.int32),
            pltpu.SemaphoreType.DMA,
            pltpu.SemaphoreType.DMA,
            pltpu.SemaphoreType.DMA,
            pltpu.SemaphoreType.DMA,
            pltpu.SemaphoreType.DMA,
        ],
    )
    return run(feats, segment_ids, bounds)
